# Initial kernel scaffold; baseline (speedup 1.0000x reference)
import functools

import jax
import jax.numpy as jnp
from jax import lax
from jax.experimental import pallas as pl
from jax.experimental.pallas import tpu as pltpu
from jax.experimental.pallas import tpu_sc as plsc

NUM_ROWS = 1_000_000
DIM = 64
BATCH = 16384
HIST = 200
PACK = 16
HALF = HIST // 2

NC = 2
NS = 16
NW = NC * NS
BPW = BATCH // NW
CHUNK = 128
NCHUNKS = BPW // CHUNK


def _tc_matvec(table, wc):
    rb = 8000

    def body(t_ref, w_ref, o_ref):
        o_ref[...] = jnp.dot(t_ref[...], w_ref[...],
                             preferred_element_type=jnp.float32)

    return pl.pallas_call(
        body,
        grid=(NUM_ROWS // rb,),
        in_specs=[
            pl.BlockSpec((rb, DIM), lambda i: (i, 0)),
            pl.BlockSpec((DIM, PACK), lambda i: (0, 0)),
        ],
        out_specs=pl.BlockSpec((rb, PACK), lambda i: (i, 0)),
        out_shape=jax.ShapeDtypeStruct((NUM_ROWS, PACK), jnp.float32),
    )(table, wc)


def _sc_gather_reduce(x3, packed, bias_vec):
    mesh = plsc.VectorSubcoreMesh(core_axis_name="c", subcore_axis_name="s",
                                  num_cores=NC, num_subcores=NS)

    @functools.partial(
        pl.kernel,
        out_type=jax.ShapeDtypeStruct((BATCH, PACK), jnp.float32),
        mesh=mesh,
        scratch_types=[
            pltpu.VMEM((CHUNK, 2, HALF), jnp.int32),
            pltpu.VMEM((2, HALF, PACK), jnp.float32),
            pltpu.VMEM((CHUNK, PACK), jnp.float32),
            pltpu.VMEM((PACK,), jnp.float32),
            pltpu.SemaphoreType.DMA,
        ],
    )
    def body(x_hbm, packed_hbm, bias_hbm, out_hbm,
             idx_v, rows_v, outb_v, bias_v, gsem):
        wid = lax.axis_index("s") * NC + lax.axis_index("c")
        base = wid * BPW
        pltpu.sync_copy(bias_hbm, bias_v)

        def chunk_body(ci, _):
            cbase = base + ci * CHUNK
            pltpu.sync_copy(x_hbm.at[pl.ds(cbase, CHUNK)], idx_v)

            def batch_body(i, _):
                cp0 = pltpu.async_copy(packed_hbm.at[idx_v.at[i, 0]],
                                       rows_v.at[0], gsem)
                cp1 = pltpu.async_copy(packed_hbm.at[idx_v.at[i, 1]],
                                       rows_v.at[1], gsem)
                cp0.wait()
                cp1.wait()

                def red(j, acc):
                    return acc + rows_v[0, j] + rows_v[1, j]

                acc = lax.fori_loop(0, HALF, red,
                                    jnp.zeros((PACK,), jnp.float32))
                outb_v[i] = acc + bias_v[...]
                return 0

            lax.fori_loop(0, CHUNK, batch_body, 0)
            pltpu.sync_copy(outb_v, out_hbm.at[pl.ds(cbase, CHUNK)])
            return 0

        lax.fori_loop(0, NCHUNKS, chunk_body, 0)

    return body(x3, packed, bias_vec)


@jax.jit
def kernel(x, table, W1, b1, W2, b2):
    wc = jnp.concatenate(
        [W1.T, W2.T, jnp.zeros((DIM, PACK - 2), jnp.float32)],
        axis=1) * (1.0 / HIST)
    bias_vec = jnp.concatenate(
        [b1, b2, jnp.zeros((PACK - 2,), jnp.float32)])
    packed = _tc_matvec(table, wc)
    x3 = x.astype(jnp.int32).reshape(BATCH, 2, HALF)
    res = _sc_gather_reduce(x3, packed, bias_vec)
    return (res[:, 0], res[:, 1])


# speedup vs baseline: 1.8779x; 1.8779x over previous
import functools

import jax
import jax.numpy as jnp
from jax import lax
from jax.experimental import pallas as pl
from jax.experimental.pallas import tpu as pltpu
from jax.experimental.pallas import tpu_sc as plsc

NUM_ROWS = 1_000_000
DIM = 64
BATCH = 16384
HIST = 200
PACK = 16
HALF = HIST // 2

NC = 2
NS = 16
NW = NC * NS
BPW = BATCH // NW
CHUNK = 128
NCHUNKS = BPW // CHUNK


def _tc_matvec(table, wc):
    rb = 8000

    def body(t_ref, w_ref, o_ref):
        o_ref[...] = jnp.dot(t_ref[...], w_ref[...],
                             preferred_element_type=jnp.float32)

    return pl.pallas_call(
        body,
        grid=(NUM_ROWS // rb,),
        in_specs=[
            pl.BlockSpec((rb, DIM), lambda i: (i, 0)),
            pl.BlockSpec((DIM, PACK), lambda i: (0, 0)),
        ],
        out_specs=pl.BlockSpec((rb, PACK), lambda i: (i, 0)),
        out_shape=jax.ShapeDtypeStruct((NUM_ROWS, PACK), jnp.float32),
    )(table, wc)


def _sc_gather_reduce(x3, packed, bias_vec):
    mesh = plsc.VectorSubcoreMesh(core_axis_name="c", subcore_axis_name="s",
                                  num_cores=NC, num_subcores=NS)

    @functools.partial(
        pl.kernel,
        out_type=jax.ShapeDtypeStruct((BATCH, PACK), jnp.float32),
        mesh=mesh,
        scratch_types=[
            pltpu.VMEM((CHUNK, 2, HALF), jnp.int32),
            pltpu.VMEM((2, HALF, PACK), jnp.float32),
            pltpu.VMEM((CHUNK, PACK), jnp.float32),
            pltpu.VMEM((PACK,), jnp.float32),
            pltpu.SemaphoreType.DMA,
        ],
        compiler_params=pltpu.CompilerParams(use_tc_tiling_on_sc=False),
    )
    def body(x_hbm, packed_hbm, bias_hbm, out_hbm,
             idx_v, rows_v, outb_v, bias_v, gsem):
        wid = lax.axis_index("s") * NC + lax.axis_index("c")
        base = wid * BPW
        pltpu.sync_copy(bias_hbm, bias_v)

        def chunk_body(ci, _):
            cbase = base + ci * CHUNK
            pltpu.sync_copy(x_hbm.at[pl.ds(cbase, CHUNK)], idx_v)

            def batch_body(i, _):
                cp0 = pltpu.async_copy(packed_hbm.at[idx_v.at[i, 0]],
                                       rows_v.at[0], gsem)
                cp1 = pltpu.async_copy(packed_hbm.at[idx_v.at[i, 1]],
                                       rows_v.at[1], gsem)
                cp0.wait()
                cp1.wait()

                def red(j, acc):
                    return acc + rows_v[0, j] + rows_v[1, j]

                acc = lax.fori_loop(0, HALF, red,
                                    jnp.zeros((PACK,), jnp.float32))
                outb_v[i] = acc + bias_v[...]
                return 0

            lax.fori_loop(0, CHUNK, batch_body, 0)
            pltpu.sync_copy(outb_v, out_hbm.at[pl.ds(cbase, CHUNK)])
            return 0

        lax.fori_loop(0, NCHUNKS, chunk_body, 0)

    return body(x3, packed, bias_vec)


@jax.jit
def kernel(x, table, W1, b1, W2, b2):
    wc = jnp.concatenate(
        [W1.T, W2.T, jnp.zeros((DIM, PACK - 2), jnp.float32)],
        axis=1) * (1.0 / HIST)
    bias_vec = jnp.concatenate(
        [b1, b2, jnp.zeros((PACK - 2,), jnp.float32)])
    packed = _tc_matvec(table, wc)
    x3 = x.astype(jnp.int32).reshape(BATCH, 2, HALF)
    res = _sc_gather_reduce(x3, packed, bias_vec)
    return (res[:, 0], res[:, 1])


# speedup vs baseline: 3.6395x; 1.9380x over previous
import functools

import jax
import jax.numpy as jnp
from jax import lax
from jax.experimental import pallas as pl
from jax.experimental.pallas import tpu as pltpu
from jax.experimental.pallas import tpu_sc as plsc

NUM_ROWS = 1_000_000
DIM = 64
BATCH = 16384
HIST = 200
HALF = HIST // 2
PADH = 104
BUF = 2 * PADH
NVR = BUF // 16

NC = 2
NS = 16
NW = NC * NS
BPW = BATCH // NW
CHUNK = 256
NCHUNKS = BPW // CHUNK
NBUF = 16
NGROUPS = CHUNK // NBUF


def _tc_heads(table, w12):
    rb = 8192
    grid = pl.cdiv(NUM_ROWS, rb)

    def body(t_ref, w_ref, o1_ref, o2_ref):
        r = lax.dot_general(w_ref[...], t_ref[...], (((1,), (1,)), ((), ())),
                            preferred_element_type=jnp.float32)
        o1_ref[...] = r[0]
        o2_ref[...] = r[1]

    return pl.pallas_call(
        body,
        grid=(grid,),
        in_specs=[
            pl.BlockSpec((rb, DIM), lambda i: (i, 0)),
            pl.BlockSpec((2, DIM), lambda i: (0, 0)),
        ],
        out_specs=[
            pl.BlockSpec((rb,), lambda i: (i,)),
            pl.BlockSpec((rb,), lambda i: (i,)),
        ],
        out_shape=[
            jax.ShapeDtypeStruct((NUM_ROWS,), jnp.float32),
            jax.ShapeDtypeStruct((NUM_ROWS,), jnp.float32),
        ],
    )(table, w12)


def _sc_gather_reduce(x3, p1, p2, bias_vec):
    mesh = plsc.VectorSubcoreMesh(core_axis_name="c", subcore_axis_name="s",
                                  num_cores=NC, num_subcores=NS)

    @functools.partial(
        pl.kernel,
        out_type=[jax.ShapeDtypeStruct((BATCH,), jnp.float32),
                  jax.ShapeDtypeStruct((BATCH,), jnp.float32)],
        mesh=mesh,
        scratch_types=[
            pltpu.VMEM((CHUNK, 2, HALF), jnp.int32),
            pltpu.VMEM((NBUF, BUF), jnp.float32),
            pltpu.VMEM((NBUF, BUF), jnp.float32),
            pltpu.VMEM((CHUNK,), jnp.float32),
            pltpu.VMEM((CHUNK,), jnp.float32),
            pltpu.VMEM((16,), jnp.float32),
            pltpu.SemaphoreType.DMA((NBUF,)),
        ],
        compiler_params=pltpu.CompilerParams(use_tc_tiling_on_sc=False,
                                             needs_layout_passes=False),
    )
    def body(x_hbm, p1_hbm, p2_hbm, bias_hbm, out1_hbm, out2_hbm,
             idx_v, buf1_v, buf2_v, o1_v, o2_v, bias_v, sems):
        wid = lax.axis_index("s") * NC + lax.axis_index("c")
        base = wid * BPW
        pltpu.sync_copy(bias_hbm, bias_v)
        bv = bias_v[...]
        b1s = bv[0]
        b2s = bv[1]
        lanes = lax.iota(jnp.int32, 16)

        zeros16 = jnp.broadcast_to(jnp.float32(0.0), (16,))
        for s in range(NBUF):
            for j in range(NVR):
                buf1_v[s, pl.ds(16 * j, 16)] = zeros16
                buf2_v[s, pl.ds(16 * j, 16)] = zeros16

        def gathers(li, s):
            return [
                (p1_hbm.at[idx_v.at[li, 0]], buf1_v.at[s].at[pl.ds(0, HALF)]),
                (p1_hbm.at[idx_v.at[li, 1]], buf1_v.at[s].at[pl.ds(PADH, HALF)]),
                (p2_hbm.at[idx_v.at[li, 0]], buf2_v.at[s].at[pl.ds(0, HALF)]),
                (p2_hbm.at[idx_v.at[li, 1]], buf2_v.at[s].at[pl.ds(PADH, HALF)]),
            ]

        def issue(li, s):
            for src, dst in gathers(li, s):
                pltpu.async_copy(src, dst, sems.at[s])

        def drain(li, s):
            for src, dst in gathers(li, s):
                pltpu.make_async_copy(src, dst, sems.at[s]).wait()

        def reduce(s, v1, v2):
            acc1 = buf1_v[s, pl.ds(0, 16)]
            acc2 = buf2_v[s, pl.ds(0, 16)]
            for j in range(1, NVR):
                acc1 = acc1 + buf1_v[s, pl.ds(16 * j, 16)]
                acc2 = acc2 + buf2_v[s, pl.ds(16 * j, 16)]
            s1 = jnp.sum(acc1) + b1s
            s2 = jnp.sum(acc2) + b2s
            sel = lanes == s
            v1 = jnp.where(sel, jnp.broadcast_to(s1, (16,)), v1)
            v2 = jnp.where(sel, jnp.broadcast_to(s2, (16,)), v2)
            return v1, v2

        def chunk_body(ci, _):
            cbase = base + ci * CHUNK
            pltpu.sync_copy(x_hbm.at[pl.ds(cbase, CHUNK)], idx_v)
            for b in range(NBUF):
                issue(b, b)

            def group_body(g, _):
                v1 = zeros16
                v2 = zeros16
                for b in range(NBUF):
                    li = g * NBUF + b
                    drain(li, b)
                    v1, v2 = reduce(b, v1, v2)

                    @pl.when(li + NBUF < CHUNK)
                    def _():
                        issue(li + NBUF, b)
                o1_v[pl.ds(g * NBUF, 16)] = v1
                o2_v[pl.ds(g * NBUF, 16)] = v2
                return 0

            lax.fori_loop(0, NGROUPS, group_body, 0)
            pltpu.sync_copy(o1_v, out1_hbm.at[pl.ds(cbase, CHUNK)])
            pltpu.sync_copy(o2_v, out2_hbm.at[pl.ds(cbase, CHUNK)])
            return 0

        lax.fori_loop(0, NCHUNKS, chunk_body, 0)

    return body(x3, p1, p2, bias_vec)


@jax.jit
def kernel(x, table, W1, b1, W2, b2):
    w12 = jnp.concatenate([W1, W2], axis=0) * (1.0 / HIST)
    bias_vec = jnp.concatenate(
        [b1, b2, jnp.zeros((14,), jnp.float32)])
    p1, p2 = _tc_heads(table, w12)
    x3 = x.astype(jnp.int32).reshape(BATCH, 2, HALF)
    out1, out2 = _sc_gather_reduce(x3, p1, p2, bias_vec)
    return (out1, out2)
